# baseline (speedup 1.0000x reference)
import functools

import jax
import jax.numpy as jnp
from jax import lax
from jax.experimental import pallas as pl
from jax.experimental.pallas import tpu as pltpu
from jax.experimental.pallas import tpu_sc as plsc

N_NODES = 10000
D = 128
N_EDGES = 320000
EPS = 1e-5

NC = 2
NS = 16
NW = NC * NS

CHUNK = 128
NCHUNK = 81
EDGES_PER_TILE = NCHUNK * CHUNK
E_PAD = EDGES_PER_TILE * NW
ACC_ROWS = 10080
ZROWS = ACC_ROWS // NS
OUT_PER_TILE = (N_NODES // NS) // 8 * 8
OUT_TAIL = N_NODES - NS * OUT_PER_TILE


NBUF = 3
ZCP = 126


def _sc_agg_body(h_hbm, src_hbm, dst_hbm, out_hbm, acc_sh, srcs, dsts,
                 rows, semg, sems):
    cid = lax.axis_index("c")
    sid = lax.axis_index("s")
    wid = cid * NS + sid

    zv = jnp.zeros((16,), jnp.float32)

    def zbuf(t, _):
        i = t // (D // 16)
        j = t % (D // 16)
        rows[0][i, pl.ds(j * 16, 16)] = zv
        return 0

    lax.fori_loop(0, CHUNK * (D // 16), zbuf, 0)

    def zcp(k, _):
        pltpu.sync_copy(rows[0].at[pl.ds(0, ZCP)],
                        acc_sh.at[pl.ds(sid * ZROWS + k * ZCP, ZCP)])
        return 0

    lax.fori_loop(0, ZROWS // ZCP, zcp, 0)
    plsc.subcore_barrier()

    base = wid * EDGES_PER_TILE

    for b in range(NBUF):
        pltpu.sync_copy(src_hbm.at[pl.ds(base + b * CHUNK, CHUNK)], srcs[b])
        pltpu.async_copy(h_hbm.at[srcs[b]], rows[b], semg[b])

    def step(i, _):
        c0 = i * NBUF
        for b in range(NBUF):
            off = base + (c0 + b) * CHUNK
            pltpu.make_async_copy(h_hbm.at[srcs[b]], rows[b], semg[b]).wait()
            pltpu.sync_copy(dst_hbm.at[pl.ds(off, CHUNK)], dsts[b])
            pltpu.async_copy(rows[b], acc_sh.at[dsts[b]], sems[b], add=True)
        for b in range(NBUF):
            c = c0 + b

            @pl.when(c < NCHUNK - NBUF)
            def _refill():
                pltpu.make_async_copy(rows[b], acc_sh.at[dsts[b]],
                                      sems[b]).wait()
                off2 = base + (c + NBUF) * CHUNK
                pltpu.sync_copy(src_hbm.at[pl.ds(off2, CHUNK)], srcs[b])
                pltpu.async_copy(h_hbm.at[srcs[b]], rows[b], semg[b])

        return 0

    lax.fori_loop(0, NCHUNK // NBUF, step, 0)
    for b in range(NBUF):
        pltpu.make_async_copy(rows[b], acc_sh.at[dsts[b]], sems[b]).wait()
    plsc.subcore_barrier()

    pltpu.sync_copy(acc_sh.at[pl.ds(sid * OUT_PER_TILE, OUT_PER_TILE)],
                    out_hbm.at[cid, pl.ds(sid * OUT_PER_TILE, OUT_PER_TILE)])

    @pl.when(sid == 0)
    def _tail():
        pltpu.sync_copy(acc_sh.at[pl.ds(NS * OUT_PER_TILE, OUT_TAIL)],
                        out_hbm.at[cid, pl.ds(NS * OUT_PER_TILE, OUT_TAIL)])


@functools.cache
def _make_sc_agg():
    return pl.kernel(
        _sc_agg_body,
        out_type=jax.ShapeDtypeStruct((NC, N_NODES, D), jnp.float32),
        mesh=plsc.VectorSubcoreMesh(core_axis_name="c", subcore_axis_name="s",
                                    num_cores=NC, num_subcores=NS),
        scratch_types=[
            pltpu.VMEM_SHARED((ACC_ROWS, D), jnp.float32),
            tuple(pltpu.VMEM((CHUNK,), jnp.int32) for _ in range(NBUF)),
            tuple(pltpu.VMEM((CHUNK,), jnp.int32) for _ in range(NBUF)),
            tuple(pltpu.VMEM((CHUNK, D), jnp.float32) for _ in range(NBUF)),
            tuple(pltpu.SemaphoreType.DMA for _ in range(NBUF)),
            tuple(pltpu.SemaphoreType.DMA for _ in range(NBUF)),
        ],
    )


def _mm_body(x_ref, w_ref, o_ref):
    o_ref[...] = jnp.dot(x_ref[...], w_ref[...],
                         preferred_element_type=jnp.float32)


_mm = pl.pallas_call(
    _mm_body,
    out_shape=jax.ShapeDtypeStruct((N_NODES, D), jnp.float32),
)


def _bn_mm_body(a_ref, b_ref, g_ref, bt_ref, w_ref, o_ref):
    t = a_ref[0] + a_ref[1] + b_ref[...]
    mean = jnp.mean(t, axis=0, keepdims=True)
    d = t - mean
    var = jnp.mean(d * d, axis=0, keepdims=True)
    xhat = d * lax.rsqrt(var + EPS)
    y = jnp.maximum(xhat * g_ref[...] + bt_ref[...], 0.0)
    o_ref[...] = jnp.dot(y, w_ref[...], preferred_element_type=jnp.float32)


_bn_mm = pl.pallas_call(
    _bn_mm_body,
    out_shape=jax.ShapeDtypeStruct((N_NODES, D), jnp.float32),
)


def _final_body(a_ref, b_ref, o_ref):
    o_ref[...] = a_ref[0] + a_ref[1] + b_ref[...]


_final = pl.pallas_call(
    _final_body,
    out_shape=jax.ShapeDtypeStruct((N_NODES, D), jnp.float32),
)


@jax.jit
def kernel(nf_mat, conv_mat, W1, b1, g1, bt1, W2, b2, g2, bt2, W3, b3):
    src = conv_mat[0].astype(jnp.int32)
    dst = conv_mat[1].astype(jnp.int32)
    pad = E_PAD - N_EDGES
    src_p = jnp.concatenate([src, jnp.zeros((pad,), jnp.int32)])
    dst_p = jnp.concatenate([dst, jnp.full((pad,), N_NODES, jnp.int32)])
    b1r = b1.reshape(1, D)
    b3r = b3.reshape(1, D)
    b2r = b2.reshape(1, D)
    g1r = g1.reshape(1, D)
    g2r = g2.reshape(1, D)
    bt1r = bt1.reshape(1, D)
    bt2r = bt2.reshape(1, D)

    sc_agg = _make_sc_agg()
    h = _mm(nf_mat, W1)
    a = sc_agg(h, src_p, dst_p)
    h = _bn_mm(a, b1r, g1r, bt1r, W2)
    a = sc_agg(h, src_p, dst_p)
    h = _bn_mm(a, b2r, g2r, bt2r, W3)
    a = sc_agg(h, src_p, dst_p)
    return _final(a, b3r)


# speedup vs baseline: 1.0406x; 1.0406x over previous
import functools

import jax
import jax.numpy as jnp
from jax import lax
from jax.experimental import pallas as pl
from jax.experimental.pallas import tpu as pltpu
from jax.experimental.pallas import tpu_sc as plsc

N_NODES = 10000
D = 128
N_EDGES = 320000
EPS = 1e-5

NC = 2
NS = 16
NW = NC * NS

CHUNK = 128
NCHUNK = 81
EDGES_PER_TILE = NCHUNK * CHUNK
E_PAD = EDGES_PER_TILE * NW
ACC_ROWS = 10080
ZROWS = ACC_ROWS // NS
OUT_PER_TILE = (N_NODES // NS) // 8 * 8
OUT_TAIL = N_NODES - NS * OUT_PER_TILE


NBUF = 3
ZCP = 126


def _sc_agg_body(h_hbm, src_hbm, dst_hbm, out_hbm, acc_sh, srcs, dsts,
                 rows, semg):
    cid = lax.axis_index("c")
    sid = lax.axis_index("s")
    wid = cid * NS + sid

    zv = jnp.zeros((16,), jnp.float32)

    def zbuf(t, _):
        i = t // (D // 16)
        j = t % (D // 16)
        rows[0][i, pl.ds(j * 16, 16)] = zv
        return 0

    lax.fori_loop(0, CHUNK * (D // 16), zbuf, 0)

    def zcp(k, _):
        pltpu.sync_copy(rows[0].at[pl.ds(0, ZCP)],
                        acc_sh.at[pl.ds(sid * ZROWS + k * ZCP, ZCP)])
        return 0

    lax.fori_loop(0, ZROWS // ZCP, zcp, 0)
    plsc.subcore_barrier()

    base = wid * EDGES_PER_TILE

    for b in range(NBUF):
        pltpu.sync_copy(src_hbm.at[pl.ds(base + b * CHUNK, CHUNK)], srcs[b])
        pltpu.async_copy(h_hbm.at[srcs[b]], rows[b], semg[b])

    def step(i, _):
        for b in range(NBUF):
            c = i * NBUF + b
            off = base + c * CHUNK
            pltpu.make_async_copy(h_hbm.at[srcs[b]], rows[b], semg[b]).wait()
            pltpu.sync_copy(dst_hbm.at[pl.ds(off, CHUNK)], dsts[b])
            pltpu.sync_copy(rows[b], acc_sh.at[dsts[b]], add=True)

            @pl.when(c < NCHUNK - NBUF)
            def _refill():
                pltpu.sync_copy(
                    src_hbm.at[pl.ds(off + NBUF * CHUNK, CHUNK)], srcs[b])
                pltpu.async_copy(h_hbm.at[srcs[b]], rows[b], semg[b])

        return 0

    lax.fori_loop(0, NCHUNK // NBUF, step, 0)
    plsc.subcore_barrier()

    pltpu.sync_copy(acc_sh.at[pl.ds(sid * OUT_PER_TILE, OUT_PER_TILE)],
                    out_hbm.at[cid, pl.ds(sid * OUT_PER_TILE, OUT_PER_TILE)])

    @pl.when(sid == 0)
    def _tail():
        pltpu.sync_copy(acc_sh.at[pl.ds(NS * OUT_PER_TILE, OUT_TAIL)],
                        out_hbm.at[cid, pl.ds(NS * OUT_PER_TILE, OUT_TAIL)])


@functools.cache
def _make_sc_agg():
    return pl.kernel(
        _sc_agg_body,
        out_type=jax.ShapeDtypeStruct((NC, N_NODES, D), jnp.float32),
        mesh=plsc.VectorSubcoreMesh(core_axis_name="c", subcore_axis_name="s",
                                    num_cores=NC, num_subcores=NS),
        scratch_types=[
            pltpu.VMEM_SHARED((ACC_ROWS, D), jnp.float32),
            tuple(pltpu.VMEM((CHUNK,), jnp.int32) for _ in range(NBUF)),
            tuple(pltpu.VMEM((CHUNK,), jnp.int32) for _ in range(NBUF)),
            tuple(pltpu.VMEM((CHUNK, D), jnp.float32) for _ in range(NBUF)),
            tuple(pltpu.SemaphoreType.DMA for _ in range(NBUF)),
        ],
    )


def _mm_body(x_ref, w_ref, o_ref):
    o_ref[...] = jnp.dot(x_ref[...], w_ref[...],
                         preferred_element_type=jnp.float32)


_mm = pl.pallas_call(
    _mm_body,
    out_shape=jax.ShapeDtypeStruct((N_NODES, D), jnp.float32),
)


def _bn_mm_body(a_ref, b_ref, g_ref, bt_ref, w_ref, o_ref):
    t = a_ref[0] + a_ref[1] + b_ref[...]
    mean = jnp.mean(t, axis=0, keepdims=True)
    d = t - mean
    var = jnp.mean(d * d, axis=0, keepdims=True)
    xhat = d * lax.rsqrt(var + EPS)
    y = jnp.maximum(xhat * g_ref[...] + bt_ref[...], 0.0)
    o_ref[...] = jnp.dot(y, w_ref[...], preferred_element_type=jnp.float32)


_bn_mm = pl.pallas_call(
    _bn_mm_body,
    out_shape=jax.ShapeDtypeStruct((N_NODES, D), jnp.float32),
)


def _final_body(a_ref, b_ref, o_ref):
    o_ref[...] = a_ref[0] + a_ref[1] + b_ref[...]


_final = pl.pallas_call(
    _final_body,
    out_shape=jax.ShapeDtypeStruct((N_NODES, D), jnp.float32),
)


@jax.jit
def kernel(nf_mat, conv_mat, W1, b1, g1, bt1, W2, b2, g2, bt2, W3, b3):
    src = conv_mat[0].astype(jnp.int32)
    dst = conv_mat[1].astype(jnp.int32)
    pad = E_PAD - N_EDGES
    src_p = jnp.concatenate([src, jnp.zeros((pad,), jnp.int32)])
    dst_p = jnp.concatenate([dst, jnp.full((pad,), N_NODES, jnp.int32)])
    b1r = b1.reshape(1, D)
    b3r = b3.reshape(1, D)
    b2r = b2.reshape(1, D)
    g1r = g1.reshape(1, D)
    g2r = g2.reshape(1, D)
    bt1r = bt1.reshape(1, D)
    bt2r = bt2.reshape(1, D)

    sc_agg = _make_sc_agg()
    h = _mm(nf_mat, W1)
    a = sc_agg(h, src_p, dst_p)
    h = _bn_mm(a, b1r, g1r, bt1r, W2)
    a = sc_agg(h, src_p, dst_p)
    h = _bn_mm(a, b2r, g2r, bt2r, W3)
    a = sc_agg(h, src_p, dst_p)
    return _final(a, b3r)


# speedup vs baseline: 2.6267x; 2.5242x over previous
import functools

import jax
import jax.numpy as jnp
from jax import lax
from jax.experimental import pallas as pl
from jax.experimental.pallas import tpu as pltpu
from jax.experimental.pallas import tpu_sc as plsc

N_NODES = 10000
D = 128
N_EDGES = 320000
EPS = 1e-5

NC = 2
NS = 16
NW = NC * NS

CHUNK = 112
NCHUNK = 90
EDGES_PER_TILE = NCHUNK * CHUNK
E_PAD = EDGES_PER_TILE * NW
ACC_ROWS = 10240
ZROWS = ACC_ROWS // NS
OUT_PER_TILE = (N_NODES // NS) // 8 * 8
OUT_TAIL = N_NODES - NS * OUT_PER_TILE


NBUF = 3
ZCP = 80


def _sc_agg_body(h_hbm, src_hbm, dst_hbm, out_hbm, acc_sh, srcs, dst_v,
                 rows, semg):
    cid = lax.axis_index("c")
    sid = lax.axis_index("s")
    wid = cid * NS + sid

    zv = jnp.zeros((16,), jnp.float32)

    def zbuf(t, _):
        i = t // (D // 16)
        j = t % (D // 16)
        rows[0][i, pl.ds(j * 16, 16)] = zv
        return 0

    lax.fori_loop(0, CHUNK * (D // 16), zbuf, 0)

    def zcp(k, _):
        pltpu.sync_copy(rows[0].at[pl.ds(0, ZCP)],
                        acc_sh.at[pl.ds(sid * ZROWS + k * ZCP, ZCP)])
        return 0

    lax.fori_loop(0, ZROWS // ZCP, zcp, 0)
    plsc.subcore_barrier()

    base = wid * EDGES_PER_TILE

    for b in range(NBUF):
        pltpu.sync_copy(src_hbm.at[pl.ds(base + b * CHUNK, CHUNK)], srcs[b])
        pltpu.async_copy(h_hbm.at[srcs[b]], rows[b], semg[b])

    def step(i, _):
        for b in range(NBUF):
            c = i * NBUF + b
            off = base + c * CHUNK
            pltpu.make_async_copy(h_hbm.at[srcs[b]], rows[b], semg[b]).wait()
            pltpu.sync_copy(dst_hbm.at[pl.ds(off, CHUNK)], dst_v)
            pltpu.sync_copy(rows[b], acc_sh.at[dst_v], add=True)

            @pl.when(c < NCHUNK - NBUF)
            def _refill():
                pltpu.sync_copy(
                    src_hbm.at[pl.ds(off + NBUF * CHUNK, CHUNK)], srcs[b])
                pltpu.async_copy(h_hbm.at[srcs[b]], rows[b], semg[b])

        return 0

    lax.fori_loop(0, NCHUNK // NBUF, step, 0)
    plsc.subcore_barrier()

    pltpu.sync_copy(acc_sh.at[pl.ds(sid * OUT_PER_TILE, OUT_PER_TILE)],
                    out_hbm.at[cid, pl.ds(sid * OUT_PER_TILE, OUT_PER_TILE)])

    @pl.when(sid == 0)
    def _tail():
        pltpu.sync_copy(acc_sh.at[pl.ds(NS * OUT_PER_TILE, OUT_TAIL)],
                        out_hbm.at[cid, pl.ds(NS * OUT_PER_TILE, OUT_TAIL)])


@functools.cache
def _make_sc_agg():
    return pl.kernel(
        _sc_agg_body,
        out_type=jax.ShapeDtypeStruct((NC, N_NODES, D), jnp.float32),
        mesh=plsc.VectorSubcoreMesh(core_axis_name="c", subcore_axis_name="s",
                                    num_cores=NC, num_subcores=NS),
        scratch_types=[
            pltpu.VMEM_SHARED((ACC_ROWS, D), jnp.float32),
            tuple(pltpu.VMEM((CHUNK,), jnp.int32) for _ in range(NBUF)),
            pltpu.VMEM((CHUNK,), jnp.int32),
            tuple(pltpu.VMEM((CHUNK, D), jnp.float32) for _ in range(NBUF)),
            tuple(pltpu.SemaphoreType.DMA for _ in range(NBUF)),
        ],
    )


def _mm_body(x_ref, w_ref, o_ref):
    o_ref[...] = jnp.dot(x_ref[...], w_ref[...],
                         preferred_element_type=jnp.float32)


_mm = pl.pallas_call(
    _mm_body,
    out_shape=jax.ShapeDtypeStruct((N_NODES, D), jnp.float32),
)


def _bn_mm_body(a_ref, b_ref, g_ref, bt_ref, w_ref, o_ref):
    t = a_ref[0] + a_ref[1] + b_ref[...]
    mean = jnp.mean(t, axis=0, keepdims=True)
    d = t - mean
    var = jnp.mean(d * d, axis=0, keepdims=True)
    xhat = d * lax.rsqrt(var + EPS)
    y = jnp.maximum(xhat * g_ref[...] + bt_ref[...], 0.0)
    o_ref[...] = jnp.dot(y, w_ref[...], preferred_element_type=jnp.float32)


_bn_mm = pl.pallas_call(
    _bn_mm_body,
    out_shape=jax.ShapeDtypeStruct((N_NODES, D), jnp.float32),
)


def _final_body(a_ref, b_ref, o_ref):
    o_ref[...] = a_ref[0] + a_ref[1] + b_ref[...]


_final = pl.pallas_call(
    _final_body,
    out_shape=jax.ShapeDtypeStruct((N_NODES, D), jnp.float32),
)


@jax.jit
def kernel(nf_mat, conv_mat, W1, b1, g1, bt1, W2, b2, g2, bt2, W3, b3):
    src = conv_mat[0].astype(jnp.int32)
    dst = conv_mat[1].astype(jnp.int32)
    pad = E_PAD - N_EDGES
    src_p = jnp.concatenate([src, jnp.zeros((pad,), jnp.int32)])
    dst_p = jnp.concatenate([dst, jnp.full((pad,), N_NODES, jnp.int32)])
    b1r = b1.reshape(1, D)
    b3r = b3.reshape(1, D)
    b2r = b2.reshape(1, D)
    g1r = g1.reshape(1, D)
    g2r = g2.reshape(1, D)
    bt1r = bt1.reshape(1, D)
    bt2r = bt2.reshape(1, D)

    sc_agg = _make_sc_agg()
    h = _mm(nf_mat, W1)
    a = sc_agg(h, src_p, dst_p)
    h = _bn_mm(a, b1r, g1r, bt1r, W2)
    a = sc_agg(h, src_p, dst_p)
    h = _bn_mm(a, b2r, g2r, bt2r, W3)
    a = sc_agg(h, src_p, dst_p)
    return _final(a, b3r)


# speedup vs baseline: 2.6616x; 1.0133x over previous
import functools

import jax
import jax.numpy as jnp
from jax import lax
from jax.experimental import pallas as pl
from jax.experimental.pallas import tpu as pltpu
from jax.experimental.pallas import tpu_sc as plsc

N_NODES = 10000
D = 128
N_EDGES = 320000
EPS = 1e-5

NC = 2
NS = 16
NW = NC * NS

CHUNK = 120
NCHUNK = 84
EDGES_PER_TILE = NCHUNK * CHUNK
E_PAD = EDGES_PER_TILE * NW
ACC_ROWS = 10240
ZROWS = ACC_ROWS // NS
OUT_PER_TILE = (N_NODES // NS) // 8 * 8
OUT_TAIL = N_NODES - NS * OUT_PER_TILE


NBUF = 3
ZCP = 80


def _sc_agg_body(h_hbm, src_hbm, dst_hbm, out_hbm, acc_sh, srcs, dst_v,
                 rows, semg):
    cid = lax.axis_index("c")
    sid = lax.axis_index("s")
    wid = cid * NS + sid

    zv = jnp.zeros((16,), jnp.float32)

    def zbuf(t, _):
        i = t // (D // 16)
        j = t % (D // 16)
        rows[0][i, pl.ds(j * 16, 16)] = zv
        return 0

    lax.fori_loop(0, CHUNK * (D // 16), zbuf, 0)

    def zcp(k, _):
        pltpu.sync_copy(rows[0].at[pl.ds(0, ZCP)],
                        acc_sh.at[pl.ds(sid * ZROWS + k * ZCP, ZCP)])
        return 0

    lax.fori_loop(0, ZROWS // ZCP, zcp, 0)
    plsc.subcore_barrier()

    base = wid * EDGES_PER_TILE

    for b in range(NBUF):
        pltpu.sync_copy(src_hbm.at[pl.ds(base + b * CHUNK, CHUNK)], srcs[b])
        pltpu.async_copy(h_hbm.at[srcs[b]], rows[b], semg[b])

    def step(i, _):
        for b in range(NBUF):
            c = i * NBUF + b
            off = base + c * CHUNK
            pltpu.make_async_copy(h_hbm.at[srcs[b]], rows[b], semg[b]).wait()
            pltpu.sync_copy(dst_hbm.at[pl.ds(off, CHUNK)], dst_v)
            pltpu.sync_copy(rows[b], acc_sh.at[dst_v], add=True)

            @pl.when(c < NCHUNK - NBUF)
            def _refill():
                pltpu.sync_copy(
                    src_hbm.at[pl.ds(off + NBUF * CHUNK, CHUNK)], srcs[b])
                pltpu.async_copy(h_hbm.at[srcs[b]], rows[b], semg[b])

        return 0

    lax.fori_loop(0, NCHUNK // NBUF, step, 0)
    plsc.subcore_barrier()

    pltpu.sync_copy(acc_sh.at[pl.ds(sid * OUT_PER_TILE, OUT_PER_TILE)],
                    out_hbm.at[cid, pl.ds(sid * OUT_PER_TILE, OUT_PER_TILE)])

    @pl.when(sid == 0)
    def _tail():
        pltpu.sync_copy(acc_sh.at[pl.ds(NS * OUT_PER_TILE, OUT_TAIL)],
                        out_hbm.at[cid, pl.ds(NS * OUT_PER_TILE, OUT_TAIL)])


@functools.cache
def _make_sc_agg():
    return pl.kernel(
        _sc_agg_body,
        out_type=jax.ShapeDtypeStruct((NC, N_NODES, D), jnp.float32),
        mesh=plsc.VectorSubcoreMesh(core_axis_name="c", subcore_axis_name="s",
                                    num_cores=NC, num_subcores=NS),
        scratch_types=[
            pltpu.VMEM_SHARED((ACC_ROWS, D), jnp.float32),
            tuple(pltpu.VMEM((CHUNK,), jnp.int32) for _ in range(NBUF)),
            pltpu.VMEM((CHUNK,), jnp.int32),
            tuple(pltpu.VMEM((CHUNK, D), jnp.float32) for _ in range(NBUF)),
            tuple(pltpu.SemaphoreType.DMA for _ in range(NBUF)),
        ],
    )


def _mm_body(x_ref, w_ref, o_ref):
    o_ref[...] = jnp.dot(x_ref[...], w_ref[...],
                         preferred_element_type=jnp.float32)


_mm = pl.pallas_call(
    _mm_body,
    out_shape=jax.ShapeDtypeStruct((N_NODES, D), jnp.float32),
)


def _bn_mm_body(a_ref, b_ref, g_ref, bt_ref, w_ref, o_ref):
    t = a_ref[0] + a_ref[1] + b_ref[...]
    mean = jnp.mean(t, axis=0, keepdims=True)
    d = t - mean
    var = jnp.mean(d * d, axis=0, keepdims=True)
    xhat = d * lax.rsqrt(var + EPS)
    y = jnp.maximum(xhat * g_ref[...] + bt_ref[...], 0.0)
    o_ref[...] = jnp.dot(y, w_ref[...], preferred_element_type=jnp.float32)


_bn_mm = pl.pallas_call(
    _bn_mm_body,
    out_shape=jax.ShapeDtypeStruct((N_NODES, D), jnp.float32),
)


def _final_body(a_ref, b_ref, o_ref):
    o_ref[...] = a_ref[0] + a_ref[1] + b_ref[...]


_final = pl.pallas_call(
    _final_body,
    out_shape=jax.ShapeDtypeStruct((N_NODES, D), jnp.float32),
)


@jax.jit
def kernel(nf_mat, conv_mat, W1, b1, g1, bt1, W2, b2, g2, bt2, W3, b3):
    src = conv_mat[0].astype(jnp.int32)
    dst = conv_mat[1].astype(jnp.int32)
    pad = E_PAD - N_EDGES
    src_p = jnp.concatenate([src, jnp.zeros((pad,), jnp.int32)])
    dst_p = jnp.concatenate([dst, jnp.full((pad,), N_NODES, jnp.int32)])
    b1r = b1.reshape(1, D)
    b3r = b3.reshape(1, D)
    b2r = b2.reshape(1, D)
    g1r = g1.reshape(1, D)
    g2r = g2.reshape(1, D)
    bt1r = bt1.reshape(1, D)
    bt2r = bt2.reshape(1, D)

    sc_agg = _make_sc_agg()
    h = _mm(nf_mat, W1)
    a = sc_agg(h, src_p, dst_p)
    h = _bn_mm(a, b1r, g1r, bt1r, W2)
    a = sc_agg(h, src_p, dst_p)
    h = _bn_mm(a, b2r, g2r, bt2r, W3)
    a = sc_agg(h, src_p, dst_p)
    return _final(a, b3r)
